# baseline (speedup 1.0000x reference)
import dataclasses
import functools

import jax
import jax.numpy as jnp
from jax.experimental import pallas as pl
from jax.experimental.pallas import tpu as pltpu
from jax.experimental.pallas import tpu_sc as plsc

B, D, H, C, E, EH, EO, T = 8192, 2048, 512, 256, 16, 256, 128, 3
TILE = 2048
_NW = 32
_TPW = B // _NW
_L = 16


def _gelu(v):
    return 0.5 * v * (1.0 + jax.lax.erf(v * (2.0 ** -0.5)))


def _proj_gate_kernel(x_ref, task_ref, w1_ref, b1_ref, lng_ref, lnb_ref,
                      w2_ref, b2_ref, gw1z_ref, gw1t_ref, gb1_ref, gw2_ref,
                      gb2_ref, z_ref, gl_ref, glt_ref, gp_ref, load_ref):
    h = _gelu(jnp.dot(x_ref[...], w1_ref[...]) + b1_ref[...])
    mu = jnp.mean(h, axis=-1, keepdims=True)
    hc = h - mu
    var = jnp.mean(hc * hc, axis=-1, keepdims=True)
    hn = hc * jax.lax.rsqrt(var + 1e-5) * lng_ref[...] + lnb_ref[...]
    z = jnp.dot(hn, w2_ref[...]) + b2_ref[...]
    z_ref[...] = z

    g1 = _gelu(jnp.dot(z, gw1z_ref[...]) + jnp.dot(task_ref[...], gw1t_ref[...])
               + gb1_ref[...])
    gl = jnp.dot(g1, gw2_ref[...]) + gb2_ref[...]
    gl_ref[...] = gl
    glt_ref[...] = gl.T

    m = jnp.max(gl, axis=-1, keepdims=True)
    eg = jnp.exp(gl - m)
    gp = eg / jnp.sum(eg, axis=-1, keepdims=True)
    gp_ref[...] = gp
    load_ref[...] = jnp.sum(gp, axis=0, keepdims=True)[None]


def _route_kernel(glt_hbm, ti1_hbm, ti2_hbm, tw1_hbm, tw2_hbm,
                  gltv, ti1v, ti2v, tw1v, tw2v, sem):
    wid = jax.lax.axis_index("s") * 2 + jax.lax.axis_index("c")
    base = wid * _TPW
    pltpu.async_copy(glt_hbm.at[:, pl.ds(base, _TPW)], gltv, sem).wait()
    fmin = jnp.float32(-3.4e38)

    @pl.loop(0, _TPW // _L)
    def _(g):
        sl = pl.ds(g * _L, _L)
        vs = [gltv[e, sl] for e in range(E)]
        m1 = vs[0]
        for e in range(1, E):
            m1 = jnp.maximum(m1, vs[e])
        i1 = jnp.full((_L,), E, jnp.int32)
        for e in range(E - 1, -1, -1):
            i1 = jnp.where(vs[e] == m1, e, i1)
        m2 = jnp.full((_L,), fmin, jnp.float32)
        for e in range(E):
            m2 = jnp.maximum(m2, jnp.where(i1 == e, fmin, vs[e]))
        i2 = jnp.full((_L,), E, jnp.int32)
        for e in range(E - 1, -1, -1):
            i2 = jnp.where((vs[e] == m2) & (i1 != e), e, i2)
        e2 = jnp.exp(m2 - m1)
        w1 = 1.0 / (1.0 + e2)
        w2 = e2 * w1
        ti1v[sl] = i1
        ti2v[sl] = i2
        tw1v[sl] = w1
        tw2v[sl] = w2

    pltpu.async_copy(ti1v, ti1_hbm.at[pl.ds(base, _TPW)], sem).wait()
    pltpu.async_copy(ti2v, ti2_hbm.at[pl.ds(base, _TPW)], sem).wait()
    pltpu.async_copy(tw1v, tw1_hbm.at[pl.ds(base, _TPW)], sem).wait()
    pltpu.async_copy(tw2v, tw2_hbm.at[pl.ds(base, _TPW)], sem).wait()


def _route(glt):
    mesh = plsc.VectorSubcoreMesh(core_axis_name="c", subcore_axis_name="s")
    cp = pltpu.CompilerParams()
    if "needs_layout_passes" in pltpu.CompilerParams.__dataclass_fields__:
        cp = dataclasses.replace(cp, needs_layout_passes=False)
    k = pl.kernel(
        _route_kernel,
        out_type=(
            jax.ShapeDtypeStruct((B,), jnp.int32),
            jax.ShapeDtypeStruct((B,), jnp.int32),
            jax.ShapeDtypeStruct((B,), jnp.float32),
            jax.ShapeDtypeStruct((B,), jnp.float32),
        ),
        mesh=mesh,
        scratch_types=[
            pltpu.VMEM((E, _TPW), jnp.float32),
            pltpu.VMEM((_TPW,), jnp.int32),
            pltpu.VMEM((_TPW,), jnp.int32),
            pltpu.VMEM((_TPW,), jnp.float32),
            pltpu.VMEM((_TPW,), jnp.float32),
            pltpu.SemaphoreType.DMA,
        ],
        compiler_params=cp,
    )
    return k(glt)


def _expert_kernel(z_ref, ti_ref, tw_ref, ew1_ref, eb1_ref, ew2_ref, eb2_ref,
                   out_ref, eo_ref):
    z = z_ref[...]
    zb = z.astype(jnp.bfloat16)
    iota = jax.lax.broadcasted_iota(jnp.int32, (TILE, E), 1)
    comb = (jnp.where(iota == ti_ref[:, 0:1], tw_ref[:, 0:1], 0.0)
            + jnp.where(iota == ti_ref[:, 1:2], tw_ref[:, 1:2], 0.0))
    acc = jnp.zeros((TILE, EO), jnp.float32)
    for e in range(E):
        ehf = (jnp.dot(zb, ew1_ref[e], preferred_element_type=jnp.float32)
               + eb1_ref[:, e * EH:(e + 1) * EH])
        ehb = _gelu(ehf.astype(jnp.bfloat16))
        eo = (jnp.dot(ehb, ew2_ref[e], preferred_element_type=jnp.float32)
              + eb2_ref[:, e * EO:(e + 1) * EO])
        eo_ref[:, e, :] = eo
        acc += comb[:, e:e + 1] * eo
    out_ref[...] = acc


@functools.partial(jax.jit, static_argnums=())
def kernel(x, task_id, proj_w1, proj_b1, ln_g, ln_b, proj_w2, proj_b2,
           exp_w1, exp_b1, exp_w2, exp_b2, gate_w1, gate_b1, gate_w2, gate_b2):
    nsteps = B // TILE
    ew1b = exp_w1.astype(jnp.bfloat16)
    ew2b = exp_w2.astype(jnp.bfloat16)
    eb1 = exp_b1.reshape(1, E * EH)
    eb2 = exp_b2.reshape(1, E * EO)
    gw1z = gate_w1[:C]
    gw1t = gate_w1[C:]

    full = lambda shape: pl.BlockSpec(shape, lambda i: tuple(0 for _ in shape))
    row = lambda shape: pl.BlockSpec(shape, lambda i: (i,) + (0,) * (len(shape) - 1))

    z, gl, glt, gp, load_sums = pl.pallas_call(
        _proj_gate_kernel,
        grid=(nsteps,),
        in_specs=[
            row((TILE, D)), row((TILE, T)), full((D, H)), full((1, H)),
            full((1, H)), full((1, H)), full((H, C)), full((1, C)),
            full((C, 2 * E)), full((T, 2 * E)), full((1, 2 * E)),
            full((2 * E, E)), full((1, E)),
        ],
        out_specs=(
            row((TILE, C)), row((TILE, E)),
            pl.BlockSpec((E, TILE), lambda i: (0, i)),
            row((TILE, E)), row((1, 1, E)),
        ),
        out_shape=(
            jax.ShapeDtypeStruct((B, C), jnp.float32),
            jax.ShapeDtypeStruct((B, E), jnp.float32),
            jax.ShapeDtypeStruct((E, B), jnp.float32),
            jax.ShapeDtypeStruct((B, E), jnp.float32),
            jax.ShapeDtypeStruct((nsteps, 1, E), jnp.float32),
        ),
        compiler_params=pltpu.CompilerParams(
            dimension_semantics=("parallel",),
        ),
    )(x, task_id, proj_w1, proj_b1.reshape(1, H), ln_g.reshape(1, H),
      ln_b.reshape(1, H), proj_w2, proj_b2.reshape(1, C), gw1z, gw1t,
      gate_b1.reshape(1, 2 * E), gate_w2, gate_b2.reshape(1, E))

    ti1, ti2, tw1, tw2 = _route(glt)
    ti = jnp.stack([ti1, ti2], axis=1)
    tw = jnp.stack([tw1, tw2], axis=1)

    out, eo = pl.pallas_call(
        _expert_kernel,
        grid=(nsteps,),
        in_specs=[
            row((TILE, C)), row((TILE, 2)), row((TILE, 2)), full((E, C, EH)),
            full((1, E * EH)), full((E, EH, EO)), full((1, E * EO)),
        ],
        out_specs=(
            row((TILE, EO)), row((TILE, E, EO)),
        ),
        out_shape=(
            jax.ShapeDtypeStruct((B, EO), jnp.float32),
            jax.ShapeDtypeStruct((B, E, EO), jnp.float32),
        ),
        compiler_params=pltpu.CompilerParams(
            dimension_semantics=("parallel",),
        ),
    )(z, ti, tw, ew1b, eb1, ew2b, eb2)

    load = jnp.sum(load_sums, axis=(0, 1)) / B
    lbl = 0.01 * (jnp.var(load, ddof=1) * E)
    return (out, z, gl, gp, lbl, eo, ti, tw)


# speedup vs baseline: 1.0150x; 1.0150x over previous
import dataclasses
import functools

import jax
import jax.numpy as jnp
from jax.experimental import pallas as pl
from jax.experimental.pallas import tpu as pltpu
from jax.experimental.pallas import tpu_sc as plsc

B, D, H, C, E, EH, EO, T = 8192, 2048, 512, 256, 16, 256, 128, 3
TILE = 1024
_NW = 32
_TPW = B // _NW
_L = 16


def _gelu(v):
    return 0.5 * v * (1.0 + jax.lax.erf(v * (2.0 ** -0.5)))


def _proj_gate_kernel(x_ref, task_ref, w1_ref, b1_ref, lng_ref, lnb_ref,
                      w2_ref, b2_ref, gw1z_ref, gw1t_ref, gb1_ref, gw2_ref,
                      gb2_ref, z_ref, gl_ref, glt_ref, gp_ref, load_ref):
    h = _gelu(jnp.dot(x_ref[...], w1_ref[...]) + b1_ref[...])
    mu = jnp.mean(h, axis=-1, keepdims=True)
    hc = h - mu
    var = jnp.mean(hc * hc, axis=-1, keepdims=True)
    hn = hc * jax.lax.rsqrt(var + 1e-5) * lng_ref[...] + lnb_ref[...]
    z = jnp.dot(hn, w2_ref[...]) + b2_ref[...]
    z_ref[...] = z

    g1 = _gelu(jnp.dot(z, gw1z_ref[...]) + jnp.dot(task_ref[...], gw1t_ref[...])
               + gb1_ref[...])
    gl = jnp.dot(g1, gw2_ref[...]) + gb2_ref[...]
    gl_ref[...] = gl
    glt_ref[...] = gl.T

    m = jnp.max(gl, axis=-1, keepdims=True)
    eg = jnp.exp(gl - m)
    gp = eg / jnp.sum(eg, axis=-1, keepdims=True)
    gp_ref[...] = gp
    load_ref[...] = jnp.sum(gp, axis=0, keepdims=True)[None]


def _route_kernel(glt_hbm, ti1_hbm, ti2_hbm, tw1_hbm, tw2_hbm,
                  gltv, ti1v, ti2v, tw1v, tw2v, sem):
    wid = jax.lax.axis_index("s") * 2 + jax.lax.axis_index("c")
    base = wid * _TPW
    pltpu.async_copy(glt_hbm.at[:, pl.ds(base, _TPW)], gltv, sem).wait()
    fmin = jnp.float32(-3.4e38)

    @pl.loop(0, _TPW // _L)
    def _(g):
        sl = pl.ds(g * _L, _L)
        vs = [gltv[e, sl] for e in range(E)]
        m1 = vs[0]
        for e in range(1, E):
            m1 = jnp.maximum(m1, vs[e])
        i1 = jnp.full((_L,), E, jnp.int32)
        for e in range(E - 1, -1, -1):
            i1 = jnp.where(vs[e] == m1, e, i1)
        m2 = jnp.full((_L,), fmin, jnp.float32)
        for e in range(E):
            m2 = jnp.maximum(m2, jnp.where(i1 == e, fmin, vs[e]))
        i2 = jnp.full((_L,), E, jnp.int32)
        for e in range(E - 1, -1, -1):
            i2 = jnp.where((vs[e] == m2) & (i1 != e), e, i2)
        e2 = jnp.exp(m2 - m1)
        w1 = 1.0 / (1.0 + e2)
        w2 = e2 * w1
        ti1v[sl] = i1
        ti2v[sl] = i2
        tw1v[sl] = w1
        tw2v[sl] = w2

    pltpu.async_copy(ti1v, ti1_hbm.at[pl.ds(base, _TPW)], sem).wait()
    pltpu.async_copy(ti2v, ti2_hbm.at[pl.ds(base, _TPW)], sem).wait()
    pltpu.async_copy(tw1v, tw1_hbm.at[pl.ds(base, _TPW)], sem).wait()
    pltpu.async_copy(tw2v, tw2_hbm.at[pl.ds(base, _TPW)], sem).wait()


def _route(glt):
    mesh = plsc.VectorSubcoreMesh(core_axis_name="c", subcore_axis_name="s")
    cp = pltpu.CompilerParams()
    if "needs_layout_passes" in pltpu.CompilerParams.__dataclass_fields__:
        cp = dataclasses.replace(cp, needs_layout_passes=False)
    k = pl.kernel(
        _route_kernel,
        out_type=(
            jax.ShapeDtypeStruct((B,), jnp.int32),
            jax.ShapeDtypeStruct((B,), jnp.int32),
            jax.ShapeDtypeStruct((B,), jnp.float32),
            jax.ShapeDtypeStruct((B,), jnp.float32),
        ),
        mesh=mesh,
        scratch_types=[
            pltpu.VMEM((E, _TPW), jnp.float32),
            pltpu.VMEM((_TPW,), jnp.int32),
            pltpu.VMEM((_TPW,), jnp.int32),
            pltpu.VMEM((_TPW,), jnp.float32),
            pltpu.VMEM((_TPW,), jnp.float32),
            pltpu.SemaphoreType.DMA,
        ],
        compiler_params=cp,
    )
    return k(glt)


def _expert_kernel(z_ref, ti_ref, tw_ref, ew1_ref, eb1_ref, ew2_ref, eb2_ref,
                   out_ref, eo_ref):
    z = z_ref[...]
    zb = z.astype(jnp.bfloat16)
    iota = jax.lax.broadcasted_iota(jnp.int32, (TILE, E), 1)
    comb = (jnp.where(iota == ti_ref[:, 0:1], tw_ref[:, 0:1], 0.0)
            + jnp.where(iota == ti_ref[:, 1:2], tw_ref[:, 1:2], 0.0))
    acc = jnp.zeros((TILE, EO), jnp.float32)
    for e in range(E):
        ehf = (jnp.dot(zb, ew1_ref[e], preferred_element_type=jnp.float32)
               + eb1_ref[:, e * EH:(e + 1) * EH])
        ehb = _gelu(ehf.astype(jnp.bfloat16))
        eo = (jnp.dot(ehb, ew2_ref[e], preferred_element_type=jnp.float32)
              + eb2_ref[:, e * EO:(e + 1) * EO])
        eo_ref[:, e, :] = eo
        acc += comb[:, e:e + 1] * eo
    out_ref[...] = acc


@functools.partial(jax.jit, static_argnums=())
def kernel(x, task_id, proj_w1, proj_b1, ln_g, ln_b, proj_w2, proj_b2,
           exp_w1, exp_b1, exp_w2, exp_b2, gate_w1, gate_b1, gate_w2, gate_b2):
    nsteps = B // TILE
    ew1b = exp_w1.astype(jnp.bfloat16)
    ew2b = exp_w2.astype(jnp.bfloat16)
    eb1 = exp_b1.reshape(1, E * EH)
    eb2 = exp_b2.reshape(1, E * EO)
    gw1z = gate_w1[:C]
    gw1t = gate_w1[C:]

    full = lambda shape: pl.BlockSpec(shape, lambda i: tuple(0 for _ in shape))
    row = lambda shape: pl.BlockSpec(shape, lambda i: (i,) + (0,) * (len(shape) - 1))

    z, gl, glt, gp, load_sums = pl.pallas_call(
        _proj_gate_kernel,
        grid=(nsteps,),
        in_specs=[
            row((TILE, D)), row((TILE, T)), full((D, H)), full((1, H)),
            full((1, H)), full((1, H)), full((H, C)), full((1, C)),
            full((C, 2 * E)), full((T, 2 * E)), full((1, 2 * E)),
            full((2 * E, E)), full((1, E)),
        ],
        out_specs=(
            row((TILE, C)), row((TILE, E)),
            pl.BlockSpec((E, TILE), lambda i: (0, i)),
            row((TILE, E)), row((1, 1, E)),
        ),
        out_shape=(
            jax.ShapeDtypeStruct((B, C), jnp.float32),
            jax.ShapeDtypeStruct((B, E), jnp.float32),
            jax.ShapeDtypeStruct((E, B), jnp.float32),
            jax.ShapeDtypeStruct((B, E), jnp.float32),
            jax.ShapeDtypeStruct((nsteps, 1, E), jnp.float32),
        ),
        compiler_params=pltpu.CompilerParams(
            dimension_semantics=("parallel",),
        ),
    )(x, task_id, proj_w1, proj_b1.reshape(1, H), ln_g.reshape(1, H),
      ln_b.reshape(1, H), proj_w2, proj_b2.reshape(1, C), gw1z, gw1t,
      gate_b1.reshape(1, 2 * E), gate_w2, gate_b2.reshape(1, E))

    ti1, ti2, tw1, tw2 = _route(glt)
    ti = jnp.stack([ti1, ti2], axis=1)
    tw = jnp.stack([tw1, tw2], axis=1)

    out, eo = pl.pallas_call(
        _expert_kernel,
        grid=(nsteps,),
        in_specs=[
            row((TILE, C)), row((TILE, 2)), row((TILE, 2)), full((E, C, EH)),
            full((1, E * EH)), full((E, EH, EO)), full((1, E * EO)),
        ],
        out_specs=(
            row((TILE, EO)), row((TILE, E, EO)),
        ),
        out_shape=(
            jax.ShapeDtypeStruct((B, EO), jnp.float32),
            jax.ShapeDtypeStruct((B, E, EO), jnp.float32),
        ),
        compiler_params=pltpu.CompilerParams(
            dimension_semantics=("parallel",),
        ),
    )(z, ti, tw, ew1b, eb1, ew2b, eb2)

    load = jnp.sum(load_sums, axis=(0, 1)) / B
    lbl = 0.01 * (jnp.var(load, ddof=1) * E)
    return (out, z, gl, gp, lbl, eo, ti, tw)
